# baseline (speedup 1.0000x reference)
import functools
import math

import jax
import jax.numpy as jnp
import numpy as np
from jax import lax
from jax.experimental import pallas as pl
from jax.experimental.pallas import tpu as pltpu
from jax.experimental.pallas import tpu_sc as plsc

N = 10000
E = 320000
C = 128
M = 64

NPAD = 10112
GSZ = 40
A0 = 400
A1 = 100
E0PT = A0 * GSZ
E1PT = A1 * GSZ
CORE1_BASE = 16 * E0PT
DW = 16
ROWS_PER_TILE = NPAD // 16


def _sc_segment_sum(h, src_e, dst_e):
    mesh = plsc.VectorSubcoreMesh(core_axis_name="c", subcore_axis_name="s")

    @functools.partial(
        pl.kernel,
        mesh=mesh,
        compiler_params=pltpu.CompilerParams(needs_layout_passes=False),
        out_type=[
            jax.ShapeDtypeStruct((2, NPAD, C), jnp.float32),
            jax.ShapeDtypeStruct((2, 16, NPAD), jnp.float32),
        ],
        scratch_types=[
            pltpu.VMEM((2, GSZ), jnp.int32),
            pltpu.VMEM((2, GSZ), jnp.int32),
            pltpu.VMEM((GSZ, C), jnp.float32),
            pltpu.VMEM((GSZ, C), jnp.float32),
            pltpu.VMEM((NPAD,), jnp.float32),
            pltpu.VMEM_SHARED((NPAD, C), jnp.float32),
            pltpu.SemaphoreType.DMA,
            pltpu.SemaphoreType.DMA,
        ],
    )
    def k(h_hbm, src_hbm, dst_hbm, agg_out, deg_out,
          sa, da, rows_a, rows_b, deg_v, agg_sh, sem_a, sem_b):
        c = lax.axis_index("c")
        s = lax.axis_index("s")

        zeros16 = jnp.zeros((16,), jnp.float32)
        ones16 = jnp.ones((16,), jnp.float32)

        def zero_deg(i):
            deg_v[pl.ds(i * 16, 16)] = zeros16
        pl.loop(0, NPAD // 16)(zero_deg)

        def zero_rows(i):
            for kk in range(C // 16):
                rows_a[i, pl.ds(kk * 16, 16)] = zeros16
        pl.loop(0, GSZ)(zero_rows)
        for b in range(ROWS_PER_TILE // GSZ):
            pltpu.sync_copy(rows_a, agg_sh.at[pl.ds(s * ROWS_PER_TILE + b * GSZ, GSZ)])
        rem = ROWS_PER_TILE % GSZ
        if rem:
            pltpu.sync_copy(
                rows_a.at[pl.ds(0, rem)],
                agg_sh.at[pl.ds(s * ROWS_PER_TILE + (ROWS_PER_TILE // GSZ) * GSZ, rem)])
        plsc.subcore_barrier()

        def load_idx(e0, k):
            pltpu.sync_copy(src_hbm.at[pl.ds(e0, GSZ)], sa.at[k])
            pltpu.sync_copy(dst_hbm.at[pl.ds(e0, GSZ)], da.at[k])

        def commit(rows, k):
            pltpu.sync_copy(rows, agg_sh.at[da.at[k]], add=True)
            for kk in range(GSZ // 16):
                idx = da[k, pl.ds(kk * 16, 16)]
                plsc.addupdate_scatter(deg_v, [idx], ones16)

        def pipeline(gc, ebase):
            load_idx(ebase, 0)
            pltpu.async_copy(h_hbm.at[sa.at[0]], rows_a, sem_a)

            def body(jj):
                j = jj * 2
                load_idx(ebase + (j + 1) * GSZ, 1)
                pltpu.async_copy(h_hbm.at[sa.at[1]], rows_b, sem_b)
                pltpu.make_async_copy(h_hbm.at[sa.at[0]], rows_a, sem_a).wait()
                commit(rows_a, 0)

                @pl.when(j + 2 < gc)
                def _():
                    load_idx(ebase + (j + 2) * GSZ, 0)
                    pltpu.async_copy(h_hbm.at[sa.at[0]], rows_a, sem_a)
                pltpu.make_async_copy(h_hbm.at[sa.at[1]], rows_b, sem_b).wait()
                commit(rows_b, 1)
            pl.loop(0, gc // 2)(body)

        @pl.when(c == 0)
        def _():
            pipeline(A0, s * E0PT)

        @pl.when(c != 0)
        def _():
            pipeline(A1, CORE1_BASE + s * E1PT)

        plsc.subcore_barrier()

        pltpu.sync_copy(agg_sh.at[pl.ds(s * ROWS_PER_TILE, ROWS_PER_TILE)],
                        agg_out.at[c, pl.ds(s * ROWS_PER_TILE, ROWS_PER_TILE)])
        pltpu.sync_copy(deg_v, deg_out.at[c, s])

    return k(h, src_e, dst_e)


_BLK = 400
_NBLK = N // _BLK


def _tc1_body(u_ref, h_ref, w_ref, out_ref, acc_ref):
    j = pl.program_id(0)

    @pl.when(j == 0)
    def _():
        acc_ref[...] = jnp.zeros_like(acc_ref)

    acc_ref[...] += lax.dot_general(
        u_ref[...], h_ref[...], (((0,), (0,)), ((), ())),
        preferred_element_type=jnp.float32)

    @pl.when(j == _NBLK - 1)
    def _():
        h_hat = acc_ref[...]
        out_ref[...] = jnp.sum(h_hat[:, :, None] * w_ref[...], axis=1)


def _tc1(U0, h, W_spec):
    return pl.pallas_call(
        _tc1_body,
        grid=(_NBLK,),
        in_specs=[
            pl.BlockSpec((_BLK, M), lambda j: (j, 0)),
            pl.BlockSpec((_BLK, C), lambda j: (j, 0)),
            pl.BlockSpec((M, C, C), lambda j: (0, 0, 0)),
        ],
        out_specs=pl.BlockSpec((M, C), lambda j: (0, 0)),
        out_shape=jax.ShapeDtypeStruct((M, C), jnp.float32),
        scratch_shapes=[pltpu.VMEM((M, C), jnp.float32)],
    )(U0, h, W_spec)


def _tc2_body(h_ref, u_ref, oh_ref, agg_ref, deg_ref, lw_ref, lb_ref,
              g_ref, b_ref, out_ref):
    agg = agg_ref[0] + agg_ref[1]
    deg = jnp.sum(deg_ref[...], axis=1)
    rdeg = 1.0 / jnp.maximum(deg, 1.0)
    ag = agg * rdeg[:, None]
    local = lax.dot_general(ag, lw_ref[...], (((1,), (1,)), ((), ())),
                            preferred_element_type=jnp.float32) + lb_ref[...]
    spec = jnp.dot(u_ref[...], oh_ref[...],
                   preferred_element_type=jnp.float32)
    x = h_ref[...] + spec + local
    mu = jnp.mean(x, axis=1, keepdims=True)
    xc = x - mu
    var = jnp.mean(xc * xc, axis=1, keepdims=True)
    xn = xc * lax.rsqrt(var + 1e-5) * g_ref[...] + b_ref[...]
    out_ref[...] = 0.5 * xn * (1.0 + lax.erf(xn * (1.0 / math.sqrt(2.0))))


def _tc2(h, U0, out_hat, agg_part, deg_part, lin_W, lin_b, ln_g, ln_b):
    return pl.pallas_call(
        _tc2_body,
        grid=(_NBLK,),
        in_specs=[
            pl.BlockSpec((_BLK, C), lambda j: (j, 0)),
            pl.BlockSpec((_BLK, M), lambda j: (j, 0)),
            pl.BlockSpec((M, C), lambda j: (0, 0)),
            pl.BlockSpec((2, _BLK, C), lambda j: (0, j, 0)),
            pl.BlockSpec((_BLK, 32), lambda j: (j, 0)),
            pl.BlockSpec((C, C), lambda j: (0, 0)),
            pl.BlockSpec((1, C), lambda j: (0, 0)),
            pl.BlockSpec((1, C), lambda j: (0, 0)),
            pl.BlockSpec((1, C), lambda j: (0, 0)),
        ],
        out_specs=pl.BlockSpec((_BLK, C), lambda j: (j, 0)),
        out_shape=jax.ShapeDtypeStruct((N, C), jnp.float32),
    )(h, U0, out_hat, agg_part, deg_part, lin_W, lin_b, ln_g, ln_b)


def kernel(h, edge_index, U0, ptr, W_spec, lin_W, lin_b, ln_g, ln_b):
    del ptr
    agg_part, deg_part = _sc_segment_sum(h, edge_index[0], edge_index[1])
    deg_part = deg_part.reshape(32, NPAD)[:, :N].T

    out_hat = _tc1(U0, h, W_spec)
    return _tc2(h, U0, out_hat, agg_part, deg_part,
                lin_W, lin_b.reshape(1, C), ln_g.reshape(1, C),
                ln_b.reshape(1, C))


# speedup vs baseline: 2.0919x; 2.0919x over previous
import functools
import math

import jax
import jax.numpy as jnp
import numpy as np
from jax import lax
from jax.experimental import pallas as pl
from jax.experimental.pallas import tpu as pltpu
from jax.experimental.pallas import tpu_sc as plsc

N = 10000
E = 320000
C = 128
M = 64

NPAD = 10112
GSZ = 80
A0 = 126
A1 = 124
E0PT = A0 * GSZ
E1PT = A1 * GSZ
CORE1_BASE = 16 * E0PT
DW = 16
ROWS_PER_TILE = NPAD // 16


def _sc_segment_sum(h, src_e, dst_e):
    mesh = plsc.VectorSubcoreMesh(core_axis_name="c", subcore_axis_name="s")

    @functools.partial(
        pl.kernel,
        mesh=mesh,
        compiler_params=pltpu.CompilerParams(needs_layout_passes=False),
        out_type=[
            jax.ShapeDtypeStruct((2, NPAD, C), jnp.float32),
            jax.ShapeDtypeStruct((2, 16, NPAD), jnp.float32),
        ],
        scratch_types=[
            pltpu.VMEM((2, GSZ), jnp.int32),
            pltpu.VMEM((2, GSZ), jnp.int32),
            pltpu.VMEM((GSZ, C), jnp.float32),
            pltpu.VMEM((GSZ, C), jnp.float32),
            pltpu.VMEM((NPAD,), jnp.float32),
            pltpu.VMEM_SHARED((NPAD, C), jnp.float32),
            pltpu.SemaphoreType.DMA,
            pltpu.SemaphoreType.DMA,
        ],
    )
    def k(h_hbm, src_hbm, dst_hbm, agg_out, deg_out,
          sa, da, rows_a, rows_b, deg_v, agg_sh, sem_a, sem_b):
        c = lax.axis_index("c")
        s = lax.axis_index("s")

        zeros16 = jnp.zeros((16,), jnp.float32)
        ones16 = jnp.ones((16,), jnp.float32)

        def zero_deg(i):
            deg_v[pl.ds(i * 16, 16)] = zeros16
        pl.loop(0, NPAD // 16)(zero_deg)

        def zero_rows(i):
            for kk in range(C // 16):
                rows_a[i, pl.ds(kk * 16, 16)] = zeros16
        pl.loop(0, GSZ)(zero_rows)
        for b in range(ROWS_PER_TILE // GSZ):
            pltpu.sync_copy(rows_a, agg_sh.at[pl.ds(s * ROWS_PER_TILE + b * GSZ, GSZ)])
        rem = ROWS_PER_TILE % GSZ
        if rem:
            pltpu.sync_copy(
                rows_a.at[pl.ds(0, rem)],
                agg_sh.at[pl.ds(s * ROWS_PER_TILE + (ROWS_PER_TILE // GSZ) * GSZ, rem)])
        plsc.subcore_barrier()

        def load_idx(e0, k):
            pltpu.sync_copy(src_hbm.at[pl.ds(e0, GSZ)], sa.at[k])
            pltpu.sync_copy(dst_hbm.at[pl.ds(e0, GSZ)], da.at[k])

        def commit(rows, k):
            pltpu.sync_copy(rows, agg_sh.at[da.at[k]], add=True)
            for kk in range(GSZ // 16):
                idx = da[k, pl.ds(kk * 16, 16)]
                plsc.addupdate_scatter(deg_v, [idx], ones16)

        def pipeline(gc, ebase):
            load_idx(ebase, 0)
            pltpu.async_copy(h_hbm.at[sa.at[0]], rows_a, sem_a)

            def body(jj):
                j = jj * 2
                load_idx(ebase + (j + 1) * GSZ, 1)
                pltpu.async_copy(h_hbm.at[sa.at[1]], rows_b, sem_b)
                pltpu.make_async_copy(h_hbm.at[sa.at[0]], rows_a, sem_a).wait()
                commit(rows_a, 0)

                @pl.when(j + 2 < gc)
                def _():
                    load_idx(ebase + (j + 2) * GSZ, 0)
                    pltpu.async_copy(h_hbm.at[sa.at[0]], rows_a, sem_a)
                pltpu.make_async_copy(h_hbm.at[sa.at[1]], rows_b, sem_b).wait()
                commit(rows_b, 1)
            pl.loop(0, gc // 2)(body)

        @pl.when(c == 0)
        def _():
            pipeline(A0, s * E0PT)

        @pl.when(c != 0)
        def _():
            pipeline(A1, CORE1_BASE + s * E1PT)

        plsc.subcore_barrier()

        pltpu.sync_copy(agg_sh.at[pl.ds(s * ROWS_PER_TILE, ROWS_PER_TILE)],
                        agg_out.at[c, pl.ds(s * ROWS_PER_TILE, ROWS_PER_TILE)])
        pltpu.sync_copy(deg_v, deg_out.at[c, s])

    return k(h, src_e, dst_e)


_BLK = 400
_NBLK = N // _BLK


def _tc1_body(u_ref, h_ref, w_ref, out_ref, acc_ref):
    j = pl.program_id(0)

    @pl.when(j == 0)
    def _():
        acc_ref[...] = jnp.zeros_like(acc_ref)

    acc_ref[...] += lax.dot_general(
        u_ref[...], h_ref[...], (((0,), (0,)), ((), ())),
        preferred_element_type=jnp.float32)

    @pl.when(j == _NBLK - 1)
    def _():
        h_hat = acc_ref[...]
        out_ref[...] = jnp.sum(h_hat[:, :, None] * w_ref[...], axis=1)


def _tc1(U0, h, W_spec):
    return pl.pallas_call(
        _tc1_body,
        grid=(_NBLK,),
        in_specs=[
            pl.BlockSpec((_BLK, M), lambda j: (j, 0)),
            pl.BlockSpec((_BLK, C), lambda j: (j, 0)),
            pl.BlockSpec((M, C, C), lambda j: (0, 0, 0)),
        ],
        out_specs=pl.BlockSpec((M, C), lambda j: (0, 0)),
        out_shape=jax.ShapeDtypeStruct((M, C), jnp.float32),
        scratch_shapes=[pltpu.VMEM((M, C), jnp.float32)],
    )(U0, h, W_spec)


def _tc2_body(h_ref, u_ref, oh_ref, agg_ref, deg_ref, lw_ref, lb_ref,
              g_ref, b_ref, out_ref):
    agg = agg_ref[0] + agg_ref[1]
    deg = jnp.sum(deg_ref[...], axis=1)
    rdeg = 1.0 / jnp.maximum(deg, 1.0)
    ag = agg * rdeg[:, None]
    local = lax.dot_general(ag, lw_ref[...], (((1,), (1,)), ((), ())),
                            preferred_element_type=jnp.float32) + lb_ref[...]
    spec = jnp.dot(u_ref[...], oh_ref[...],
                   preferred_element_type=jnp.float32)
    x = h_ref[...] + spec + local
    mu = jnp.mean(x, axis=1, keepdims=True)
    xc = x - mu
    var = jnp.mean(xc * xc, axis=1, keepdims=True)
    xn = xc * lax.rsqrt(var + 1e-5) * g_ref[...] + b_ref[...]
    out_ref[...] = 0.5 * xn * (1.0 + lax.erf(xn * (1.0 / math.sqrt(2.0))))


def _tc2(h, U0, out_hat, agg_part, deg_part, lin_W, lin_b, ln_g, ln_b):
    return pl.pallas_call(
        _tc2_body,
        grid=(_NBLK,),
        in_specs=[
            pl.BlockSpec((_BLK, C), lambda j: (j, 0)),
            pl.BlockSpec((_BLK, M), lambda j: (j, 0)),
            pl.BlockSpec((M, C), lambda j: (0, 0)),
            pl.BlockSpec((2, _BLK, C), lambda j: (0, j, 0)),
            pl.BlockSpec((_BLK, 32), lambda j: (j, 0)),
            pl.BlockSpec((C, C), lambda j: (0, 0)),
            pl.BlockSpec((1, C), lambda j: (0, 0)),
            pl.BlockSpec((1, C), lambda j: (0, 0)),
            pl.BlockSpec((1, C), lambda j: (0, 0)),
        ],
        out_specs=pl.BlockSpec((_BLK, C), lambda j: (j, 0)),
        out_shape=jax.ShapeDtypeStruct((N, C), jnp.float32),
    )(h, U0, out_hat, agg_part, deg_part, lin_W, lin_b, ln_g, ln_b)


def kernel(h, edge_index, U0, ptr, W_spec, lin_W, lin_b, ln_g, ln_b):
    del ptr
    agg_part, deg_part = _sc_segment_sum(h, edge_index[0], edge_index[1])
    deg_part = deg_part.reshape(32, NPAD)[:, :N].T

    out_hat = _tc1(U0, h, W_spec)
    return _tc2(h, U0, out_hat, agg_part, deg_part,
                lin_W, lin_b.reshape(1, C), ln_g.reshape(1, C),
                ln_b.reshape(1, C))
